# baseline (speedup 1.0000x reference)
import functools

import jax
import jax.numpy as jnp
from jax import lax
from jax.experimental import pallas as pl
from jax.experimental.pallas import tpu as pltpu
from jax.experimental.pallas import tpu_sc as plsc


def _make_lookup(B, V, D):
  info = plsc.get_sparse_core_info()
  NC, NS = info.num_cores, info.num_subcores
  N0 = 480
  N1 = B // NS - N0
  split = N0 * NS

  mesh = plsc.VectorSubcoreMesh(core_axis_name="c", subcore_axis_name="s")

  @functools.partial(
      pl.kernel,
      mesh=mesh,
      out_type=jax.ShapeDtypeStruct((B, D), jnp.float32),
      scratch_types=[
          pltpu.VMEM((max(N0, N1),), jnp.int32),
          pltpu.VMEM((max(N0, N1), D), jnp.float32),
          pltpu.VMEM_SHARED((V, D), jnp.float32),
          pltpu.SemaphoreType.DMA,
          pltpu.SemaphoreType.DMA,
          pltpu.SemaphoreType.DMA,
      ],
  )
  def lookup(t_hbm, table_hbm, out_hbm, idx_v, rows_v, table_sp,
             g0s, g1s, ssem):
    c = lax.axis_index("c")
    s = lax.axis_index("s")

    @pl.when(s == 0)
    def _():
      pltpu.sync_copy(table_hbm, table_sp)

    def run(base, n):
      h = n // 2
      pltpu.sync_copy(t_hbm.at[pl.ds(base, n)], idx_v.at[pl.ds(0, n)])
      plsc.subcore_barrier()
      g0 = pltpu.async_copy(
          table_sp.at[idx_v.at[pl.ds(0, h)]], rows_v.at[pl.ds(0, h)], g0s)
      g1 = pltpu.async_copy(
          table_sp.at[idx_v.at[pl.ds(h, h)]], rows_v.at[pl.ds(h, h)], g1s)
      g0.wait()
      s0 = pltpu.async_copy(
          rows_v.at[pl.ds(0, h)], out_hbm.at[pl.ds(base, h)], ssem)
      g1.wait()
      s1 = pltpu.async_copy(
          rows_v.at[pl.ds(h, h)], out_hbm.at[pl.ds(base + h, h)], ssem)
      s0.wait()
      s1.wait()

    @pl.when(c == 0)
    def _():
      run(s * N0, N0)

    @pl.when(c == 1)
    def _():
      run(split + s * N1, N1)

  return lookup


def kernel(t, table):
  B, = t.shape
  V, D = table.shape
  lookup = _make_lookup(B, V, D)
  return lookup(t.astype(jnp.int32), table)


# speedup vs baseline: 1.0206x; 1.0206x over previous
import functools

import jax
import jax.numpy as jnp
from jax import lax
from jax.experimental import pallas as pl
from jax.experimental.pallas import tpu as pltpu
from jax.experimental.pallas import tpu_sc as plsc


def _make_lookup(B, V, D):
  info = plsc.get_sparse_core_info()
  NC, NS = info.num_cores, info.num_subcores
  N0 = 480
  N1 = B // NS - N0
  split = N0 * NS

  mesh = plsc.VectorSubcoreMesh(core_axis_name="c", subcore_axis_name="s")

  ROWS_PER_TILE = 64
  full_tiles = V // ROWS_PER_TILE
  tail_rows = V - full_tiles * ROWS_PER_TILE

  @functools.partial(
      pl.kernel,
      mesh=mesh,
      out_type=jax.ShapeDtypeStruct((B, D), jnp.float32),
      scratch_types=[
          pltpu.VMEM((max(N0, N1),), jnp.int32),
          pltpu.VMEM((max(N0, N1), D), jnp.float32),
          pltpu.VMEM_SHARED((V, D), jnp.float32),
          pltpu.SemaphoreType.DMA,
          pltpu.SemaphoreType.DMA,
          pltpu.SemaphoreType.DMA,
      ],
  )
  def lookup(t_hbm, table_hbm, out_hbm, idx_v, rows_v, table_sp,
             g0s, isem, ssem):
    c = lax.axis_index("c")
    s = lax.axis_index("s")

    def run(base, n):
      ia = pltpu.async_copy(
          t_hbm.at[pl.ds(base, n)], idx_v.at[pl.ds(0, n)], isem)

      @pl.when(s < full_tiles)
      def _():
        r = s * ROWS_PER_TILE
        pltpu.sync_copy(table_hbm.at[pl.ds(r, ROWS_PER_TILE)],
                        table_sp.at[pl.ds(r, ROWS_PER_TILE)])

      if tail_rows:
        @pl.when(s == full_tiles)
        def _():
          r = full_tiles * ROWS_PER_TILE
          pltpu.sync_copy(table_hbm.at[pl.ds(r, tail_rows)],
                          table_sp.at[pl.ds(r, tail_rows)])

      plsc.subcore_barrier()
      ia.wait()
      pltpu.async_copy(
          table_sp.at[idx_v.at[pl.ds(0, n)]],
          rows_v.at[pl.ds(0, n)], g0s).wait()
      pltpu.sync_copy(rows_v.at[pl.ds(0, n)], out_hbm.at[pl.ds(base, n)])

    @pl.when(c == 0)
    def _():
      run(s * N0, N0)

    @pl.when(c == 1)
    def _():
      run(split + s * N1, N1)

  return lookup


def kernel(t, table):
  B, = t.shape
  V, D = table.shape
  lookup = _make_lookup(B, V, D)
  return lookup(t.astype(jnp.int32), table)
